# baseline (speedup 1.0000x reference)
import functools

import jax
import jax.numpy as jnp
from jax import lax
from jax.experimental import pallas as pl
from jax.experimental.pallas import tpu as pltpu
from jax.experimental.pallas import tpu_sc as plsc

SIGMA_MIN = 0.02
SIGMA_MAX = 0.3
EPS = 1e-09

NC = 2
NS = 16
L = 16
NW = NC * NS
LW = 128
SLOT = 64


def _encoder_call(rel_id, tau, st_t, A_t, B, DIM, K):
    BPW = B // NW
    C = 128
    NCHUNK = BPW // C
    HG = DIM // L
    ORC = C * DIM // LW

    mesh = plsc.VectorSubcoreMesh(
        core_axis_name="c", subcore_axis_name="s",
        num_cores=NC, num_subcores=NS)

    f32 = jnp.float32

    @functools.partial(
        pl.kernel,
        out_type=(
            jax.ShapeDtypeStruct((B * DIM // LW, LW), f32),
            jax.ShapeDtypeStruct((B * DIM // LW, LW), f32),
            jax.ShapeDtypeStruct((B * DIM // LW, LW), f32),
        ),
        mesh=mesh,
        scratch_types=[
            pltpu.VMEM((C,), jnp.int32), pltpu.VMEM((C,), jnp.int32),
            pltpu.VMEM((C,), f32), pltpu.VMEM((C,), f32),
            pltpu.VMEM((C,), jnp.int32), pltpu.VMEM((C,), jnp.int32),
            pltpu.VMEM((C,), jnp.int32), pltpu.VMEM((C,), jnp.int32),
            pltpu.VMEM((C,), jnp.int32), pltpu.VMEM((C,), jnp.int32),
            pltpu.VMEM((C, LW), f32), pltpu.VMEM((C, LW), f32),
            pltpu.VMEM((C, LW), f32), pltpu.VMEM((C, LW), f32),
            pltpu.VMEM((C, LW), f32), pltpu.VMEM((C, LW), f32),
            pltpu.VMEM((C, L), f32),
            pltpu.VMEM((ORC, LW), f32),
            pltpu.VMEM((ORC, LW), f32),
            pltpu.VMEM((ORC, LW), f32),
            pltpu.SemaphoreType.DMA,
            pltpu.SemaphoreType.DMA,
        ],
    )
    def enc(rel_hbm, tau_hbm, st_hbm, a_hbm,
            de_hbm, dt_hbm, dp_hbm,
            idx0, idx1, tv0, tv1,
            is0, is1, ia0_0, ia0_1, ia1_0, ia1_1,
            sv0, sv1, ae0, ae1, ao0, ao1,
            gv, oe, ot, op, sem0, sem1):
        wid = lax.axis_index("s") * NC + lax.axis_index("c")
        base = wid * BPW
        bufs = ((idx0, tv0, is0, ia0_0, ia1_0, sv0, ae0, ao0, sem0),
                (idx1, tv1, is1, ia0_1, ia1_1, sv1, ae1, ao1, sem1))

        def start(c, slot):
            (idxv, tv, ist, ia0, ia1, sv, ae, ao, sem) = bufs[slot]
            off = pl.multiple_of(base + c * C, C)
            pltpu.sync_copy(rel_hbm.at[pl.ds(off, C)], idxv)
            pltpu.sync_copy(tau_hbm.at[pl.ds(off, C)], tv)

            def ibody(t, carry):
                r = idxv[pl.ds(t * L, L)]
                ist[pl.ds(t * L, L)] = lax.shift_right_logical(r, 1)
                r2 = r + r
                ia0[pl.ds(t * L, L)] = r2
                ia1[pl.ds(t * L, L)] = r2 + 1
                return carry

            lax.fori_loop(0, C // L, ibody, 0)
            return (
                pltpu.async_copy(st_hbm.at[ist], sv, sem),
                pltpu.async_copy(a_hbm.at[ia0], ae, sem),
                pltpu.async_copy(a_hbm.at[ia1], ao, sem),
            )

        def compute(slot):
            (idxv, tv, ist, ia0, ia1, sv, ae, ao, _) = bufs[slot]

            def gbody(g, carry):
                b0 = g * L
                tvec = tv[pl.ds(b0, L)]
                rvec = idxv[pl.ds(b0, L)]
                for i in range(L):
                    b = b0 + i
                    mso = (rvec[i] & 1) * SLOT + DIM
                    row = sv[b, pl.ds(mso, L)]
                    sig = SIGMA_MIN + (SIGMA_MAX - SIGMA_MIN) / (
                        1.0 + jnp.exp(-row))
                    den = 2.0 * sig * sig + EPS
                    d = tvec[i] - row
                    q = lax.rev(d * d, (0,)) / den
                    gv[b, :] = lax.rev(jnp.exp(-q), (0,))
                return carry

            lax.fori_loop(0, C // L, gbody, 0)

            def obody(g, carry):
                b0 = g * L
                tvec = tv[pl.ds(b0, L)]
                rvec = idxv[pl.ds(b0, L)]
                for i in range(L):
                    b = b0 + i
                    tau_b = tvec[i]
                    swo = (rvec[i] & 1) * SLOT
                    gvec = gv[b, :]
                    for h in range(HG):
                        fo = i * DIM + h * L
                        orow = g * (L * DIM // LW) + fo // LW
                        ocol = fo % LW
                        w16 = sv[b, pl.ds(swo + h * L, L)]
                        dth = w16 * tau_b
                        acc0 = ae[b, pl.ds(h * L, L)] * gvec[0]
                        acc1 = ae[b, pl.ds(DIM + h * L, L)] * gvec[1]
                        for k in range(2, K):
                            src = ae if k < K // 2 else ao
                            co = (k % (K // 2)) * DIM + h * L
                            a16 = src[b, pl.ds(co, L)]
                            if k % 2 == 0:
                                acc0 = acc0 + a16 * gvec[k]
                            else:
                                acc1 = acc1 + a16 * gvec[k]
                        acc = acc0 + acc1
                        ot[orow, pl.ds(ocol, L)] = dth
                        op[orow, pl.ds(ocol, L)] = acc
                        oe[orow, pl.ds(ocol, L)] = dth + acc
                return carry

            lax.fori_loop(0, C // L, obody, 0)

        pend = start(0, 0)
        for c in range(NCHUNK):
            slot = c & 1
            cur = pend
            if c + 1 < NCHUNK:
                pend = start(c + 1, 1 - slot)
            for cp in cur:
                cp.wait()
            compute(slot)
            oro = pl.multiple_of((base + c * C) * DIM // LW, ORC)
            pltpu.sync_copy(oe, de_hbm.at[pl.ds(oro, ORC)])
            pltpu.sync_copy(ot, dt_hbm.at[pl.ds(oro, ORC)])
            pltpu.sync_copy(op, dp_hbm.at[pl.ds(oro, ORC)])

    return enc(rel_id, tau, st_t, A_t)


def kernel(rel_id, tau, W_trend, A, mu, s):
    N, K, DIM = A.shape
    B = rel_id.shape[0]
    A_t = A.reshape(2 * N, LW)
    st_t = jnp.concatenate(
        [W_trend, mu, s[:, ::-1],
         jnp.zeros((N, SLOT - DIM - 2 * K), jnp.float32)],
        axis=1).reshape(N * SLOT // LW, LW)
    de, dt, dp = _encoder_call(rel_id.astype(jnp.int32), tau,
                               st_t, A_t, B, DIM, K)
    return (de.reshape(B, DIM), dt.reshape(B, DIM), dp.reshape(B, DIM))


# speedup vs baseline: 3.2944x; 3.2944x over previous
import functools

import jax
import jax.numpy as jnp
from jax import lax
from jax.experimental import pallas as pl
from jax.experimental.pallas import tpu as pltpu
from jax.experimental.pallas import tpu_sc as plsc

SIGMA_MIN = 0.02
SIGMA_MAX = 0.3
EPS = 1e-09

NC = 2
NS = 16
L = 16
NW = NC * NS
LW = 128
SLOT = 64


def _encoder_call(rel_id, tau, st_t, A_t, B, DIM, K):
    BPW = B // NW
    C = 128
    NCHUNK = BPW // C
    HG = DIM // L
    AF = K * DIM
    ORC = C * DIM // LW

    mesh = plsc.VectorSubcoreMesh(
        core_axis_name="c", subcore_axis_name="s",
        num_cores=NC, num_subcores=NS)

    f32 = jnp.float32

    @functools.partial(
        pl.kernel,
        out_type=(
            jax.ShapeDtypeStruct((B * DIM // LW, LW), f32),
            jax.ShapeDtypeStruct((B * DIM // LW, LW), f32),
            jax.ShapeDtypeStruct((B * DIM // LW, LW), f32),
        ),
        mesh=mesh,
        compiler_params=pltpu.CompilerParams(use_tc_tiling_on_sc=True),
        scratch_types=[
            pltpu.VMEM((C,), jnp.int32), pltpu.VMEM((C,), jnp.int32),
            pltpu.VMEM((C,), f32), pltpu.VMEM((C,), f32),
            pltpu.VMEM((C,), jnp.int32), pltpu.VMEM((C,), jnp.int32),
            pltpu.VMEM((C, LW), f32), pltpu.VMEM((C, LW), f32),
            pltpu.VMEM((C, AF), f32), pltpu.VMEM((C, AF), f32),
            pltpu.VMEM((C, L), f32),
            pltpu.VMEM((ORC, LW), f32),
            pltpu.VMEM((ORC, LW), f32),
            pltpu.VMEM((ORC, LW), f32),
            pltpu.SemaphoreType.DMA,
            pltpu.SemaphoreType.DMA,
        ],
    )
    def enc(rel_hbm, tau_hbm, st_hbm, a_hbm,
            de_hbm, dt_hbm, dp_hbm,
            idx0, idx1, tv0, tv1, is0, is1,
            sv0, sv1, av0, av1,
            gv, oe, ot, op, sem0, sem1):
        wid = lax.axis_index("s") * NC + lax.axis_index("c")
        base = wid * BPW
        bufs = ((idx0, tv0, is0, sv0, av0, sem0),
                (idx1, tv1, is1, sv1, av1, sem1))
        shuf = jnp.full((L,), 8, jnp.int32) + (lax.iota(jnp.int32, L) & 7)
        gdn = lax.GatherDimensionNumbers(
            offset_dims=(), collapsed_slice_dims=(0,),
            start_index_map=(0,))

        def lane_shuffle(vec, idx):
            return lax.gather(
                vec, idx[:, None], gdn, (1,),
                indices_are_sorted=False, unique_indices=False,
                mode=lax.GatherScatterMode.PROMISE_IN_BOUNDS)

        def start(c, slot):
            (idxv, tv, ist, sv, av, sem) = bufs[slot]
            off = pl.multiple_of(base + c * C, C)
            pltpu.sync_copy(rel_hbm.at[pl.ds(off, C)], idxv)
            pltpu.sync_copy(tau_hbm.at[pl.ds(off, C)], tv)

            def ibody(t, carry):
                r = idxv[pl.ds(t * L, L)]
                ist[pl.ds(t * L, L)] = lax.shift_right_logical(r, 1)
                return carry

            lax.fori_loop(0, C // L, ibody, 0)
            return (
                pltpu.async_copy(st_hbm.at[ist], sv, sem),
                pltpu.async_copy(a_hbm.at[idxv], av, sem),
            )

        def compute(slot):
            (idxv, tv, ist, sv, av, _) = bufs[slot]

            def gbody(g, carry):
                b0 = g * L
                tvec = tv[pl.ds(b0, L)]
                rvec = idxv[pl.ds(b0, L)]
                for i in range(L):
                    b = b0 + i
                    mso = (rvec[i] & 1) * SLOT + DIM
                    row = sv[b, pl.ds(mso, L)]
                    sig = SIGMA_MIN + (SIGMA_MAX - SIGMA_MIN) / (
                        1.0 + jnp.exp(-row))
                    den = lane_shuffle(2.0 * sig * sig + EPS, shuf)
                    d = tvec[i] - row
                    gv[b, :] = jnp.exp(-(d * d) / den)
                return carry

            lax.fori_loop(0, C // L, gbody, 0)

            def obody(g, carry):
                b0 = g * L
                tvec = tv[pl.ds(b0, L)]
                rvec = idxv[pl.ds(b0, L)]
                for i in range(L):
                    b = b0 + i
                    tau_b = tvec[i]
                    swo = (rvec[i] & 1) * SLOT
                    gvec = gv[b, :]
                    for h in range(HG):
                        fo = i * DIM + h * L
                        orow = g * (L * DIM // LW) + fo // LW
                        ocol = fo % LW
                        w16 = sv[b, pl.ds(swo + h * L, L)]
                        dth = w16 * tau_b
                        acc0 = av[b, pl.ds(h * L, L)] * gvec[0]
                        acc1 = av[b, pl.ds(DIM + h * L, L)] * gvec[1]
                        for k in range(2, K):
                            a16 = av[b, pl.ds(k * DIM + h * L, L)]
                            if k % 2 == 0:
                                acc0 = acc0 + a16 * gvec[k]
                            else:
                                acc1 = acc1 + a16 * gvec[k]
                        acc = acc0 + acc1
                        ot[orow, pl.ds(ocol, L)] = dth
                        op[orow, pl.ds(ocol, L)] = acc
                        oe[orow, pl.ds(ocol, L)] = dth + acc
                return carry

            lax.fori_loop(0, C // L, obody, 0)

        pend = start(0, 0)
        for c in range(NCHUNK):
            slot = c & 1
            cur = pend
            if c + 1 < NCHUNK:
                pend = start(c + 1, 1 - slot)
            for cp in cur:
                cp.wait()
            compute(slot)
            oro = pl.multiple_of((base + c * C) * DIM // LW, ORC)
            pltpu.sync_copy(oe, de_hbm.at[pl.ds(oro, ORC)])
            pltpu.sync_copy(ot, dt_hbm.at[pl.ds(oro, ORC)])
            pltpu.sync_copy(op, dp_hbm.at[pl.ds(oro, ORC)])

    return enc(rel_id, tau, st_t, A_t)


def kernel(rel_id, tau, W_trend, A, mu, s):
    N, K, DIM = A.shape
    B = rel_id.shape[0]
    A_t = A.reshape(N, K * DIM)
    st_t = jnp.concatenate(
        [W_trend, mu, s,
         jnp.zeros((N, SLOT - DIM - 2 * K), jnp.float32)],
        axis=1).reshape(N * SLOT // LW, LW)
    de, dt, dp = _encoder_call(rel_id.astype(jnp.int32), tau,
                               st_t, A_t, B, DIM, K)
    return (de.reshape(B, DIM), dt.reshape(B, DIM), dp.reshape(B, DIM))


# speedup vs baseline: 3.3427x; 1.0147x over previous
import functools

import jax
import jax.numpy as jnp
from jax import lax
from jax.experimental import pallas as pl
from jax.experimental.pallas import tpu as pltpu
from jax.experimental.pallas import tpu_sc as plsc

SIGMA_MIN = 0.02
SIGMA_MAX = 0.3
EPS = 1e-09

NC = 2
NS = 16
L = 16
NW = NC * NS
LW = 128
SLOT = 64


def _encoder_call(rel_id, tau, st_t, A_t, B, DIM, K):
    BPW = B // NW
    C = 128
    NCHUNK = BPW // C
    HG = DIM // L
    AF = K * DIM
    ORC = C * DIM // LW

    mesh = plsc.VectorSubcoreMesh(
        core_axis_name="c", subcore_axis_name="s",
        num_cores=NC, num_subcores=NS)

    f32 = jnp.float32

    @functools.partial(
        pl.kernel,
        out_type=(
            jax.ShapeDtypeStruct((B * DIM // LW, LW), f32),
            jax.ShapeDtypeStruct((B * DIM // LW, LW), f32),
            jax.ShapeDtypeStruct((B * DIM // LW, LW), f32),
        ),
        mesh=mesh,
        compiler_params=pltpu.CompilerParams(use_tc_tiling_on_sc=True),
        scratch_types=[
            pltpu.VMEM((BPW,), jnp.int32),
            pltpu.VMEM((BPW,), f32),
            pltpu.VMEM((BPW,), jnp.int32),
            pltpu.VMEM((C, LW), f32), pltpu.VMEM((C, LW), f32),
            pltpu.VMEM((C, AF), f32), pltpu.VMEM((C, AF), f32),
            pltpu.VMEM((ORC, LW), f32), pltpu.VMEM((ORC, LW), f32),
            pltpu.VMEM((ORC, LW), f32), pltpu.VMEM((ORC, LW), f32),
            pltpu.VMEM((ORC, LW), f32), pltpu.VMEM((ORC, LW), f32),
            pltpu.SemaphoreType.DMA,
            pltpu.SemaphoreType.DMA,
            pltpu.SemaphoreType.DMA,
            pltpu.SemaphoreType.DMA,
        ],
    )
    def enc(rel_hbm, tau_hbm, st_hbm, a_hbm,
            de_hbm, dt_hbm, dp_hbm,
            idxv, tv, ist,
            sv0, sv1, av0, av1,
            oe0, oe1, ot0, ot1, op0, op1,
            semg0, semg1, semo0, semo1):
        wid = lax.axis_index("s") * NC + lax.axis_index("c")
        base = pl.multiple_of(wid * BPW, BPW)
        gbufs = ((sv0, av0, semg0), (sv1, av1, semg1))
        obufs = ((oe0, ot0, op0, semo0), (oe1, ot1, op1, semo1))

        pltpu.sync_copy(rel_hbm.at[pl.ds(base, BPW)], idxv)
        pltpu.sync_copy(tau_hbm.at[pl.ds(base, BPW)], tv)

        def ibody(t, carry):
            r = idxv[pl.ds(t * L, L)]
            ist[pl.ds(t * L, L)] = lax.shift_right_logical(r, 1)
            return carry

        lax.fori_loop(0, BPW // L, ibody, 0)

        shuf = jnp.full((L,), 8, jnp.int32) + (lax.iota(jnp.int32, L) & 7)
        gdn = lax.GatherDimensionNumbers(
            offset_dims=(), collapsed_slice_dims=(0,),
            start_index_map=(0,))

        def start(c, slot):
            sv, av, sem = gbufs[slot]
            return (
                pltpu.async_copy(
                    st_hbm.at[ist.at[pl.ds(c * C, C)]], sv, sem),
                pltpu.async_copy(
                    a_hbm.at[idxv.at[pl.ds(c * C, C)]], av, sem),
            )

        def compute(c, slot, oslot):
            sv, av, _ = gbufs[slot]
            oe, ot, op, _ = obufs[oslot]

            def obody(g, carry):
                b0 = c * C + g * L
                tvec = tv[pl.ds(b0, L)]
                rvec = idxv[pl.ds(b0, L)]
                for i in range(L):
                    b = g * L + i
                    tau_b = tvec[i]
                    slot_o = (rvec[i] & 1) * SLOT
                    row = sv[b, pl.ds(slot_o + DIM, L)]
                    sig = SIGMA_MIN + (SIGMA_MAX - SIGMA_MIN) / (
                        1.0 + jnp.exp(-row))
                    den = lax.gather(
                        2.0 * sig * sig + EPS, shuf[:, None], gdn, (1,),
                        mode=lax.GatherScatterMode.PROMISE_IN_BOUNDS)
                    d = tau_b - row
                    gvec = jnp.exp(-(d * d) / den)
                    for h in range(HG):
                        fo = i * DIM + h * L
                        orow = g * (L * DIM // LW) + fo // LW
                        ocol = fo % LW
                        w16 = sv[b, pl.ds(slot_o + h * L, L)]
                        dth = w16 * tau_b
                        acc0 = av[b, pl.ds(h * L, L)] * gvec[0]
                        acc1 = av[b, pl.ds(DIM + h * L, L)] * gvec[1]
                        for k in range(2, K):
                            a16 = av[b, pl.ds(k * DIM + h * L, L)]
                            if k % 2 == 0:
                                acc0 = acc0 + a16 * gvec[k]
                            else:
                                acc1 = acc1 + a16 * gvec[k]
                        acc = acc0 + acc1
                        ot[orow, pl.ds(ocol, L)] = dth
                        op[orow, pl.ds(ocol, L)] = acc
                        oe[orow, pl.ds(ocol, L)] = dth + acc
                return carry

            lax.fori_loop(0, C // L, obody, 0)

        def flush(c, oslot):
            oe, ot, op, sem = obufs[oslot]
            oro = pl.multiple_of((base + c * C) * DIM // LW, ORC)
            return (
                pltpu.async_copy(oe, de_hbm.at[pl.ds(oro, ORC)], sem),
                pltpu.async_copy(ot, dt_hbm.at[pl.ds(oro, ORC)], sem),
                pltpu.async_copy(op, dp_hbm.at[pl.ds(oro, ORC)], sem),
            )

        pend = start(0, 0)
        oflush = [None, None]
        for c in range(NCHUNK):
            slot = c & 1
            cur = pend
            if c + 1 < NCHUNK:
                pend = start(c + 1, 1 - slot)
            for cp in cur:
                cp.wait()
            if oflush[slot] is not None:
                for cp in oflush[slot]:
                    cp.wait()
            compute(c, slot, slot)
            oflush[slot] = flush(c, slot)
        for fl in oflush:
            if fl is not None:
                for cp in fl:
                    cp.wait()

    return enc(rel_id, tau, st_t, A_t)


def kernel(rel_id, tau, W_trend, A, mu, s):
    N, K, DIM = A.shape
    B = rel_id.shape[0]
    A_t = A.reshape(N, K * DIM)
    st_t = jnp.concatenate(
        [W_trend, mu, s,
         jnp.zeros((N, SLOT - DIM - 2 * K), jnp.float32)],
        axis=1).reshape(N * SLOT // LW, LW)
    de, dt, dp = _encoder_call(rel_id.astype(jnp.int32), tau,
                               st_t, A_t, B, DIM, K)
    return (de.reshape(B, DIM), dt.reshape(B, DIM), dp.reshape(B, DIM))
